# baseline (speedup 1.0000x reference)
import functools

import jax
import jax.numpy as jnp
from jax import lax
from jax.experimental import pallas as pl
from jax.experimental.pallas import tpu as pltpu
from jax.experimental.pallas import tpu_sc as plsc

N_NODES = 10000
D = 128
NC = 2
NS = 16
NW = NC * NS
CHUNK = 56
NCHUNKS = 180
E_PAD = NW * NCHUNKS * CHUNK
N_PAD = 10112
ROWS_PER_SUB = N_PAD // NS
DEG_W = 16

_MESH = plsc.VectorSubcoreMesh(core_axis_name="c", subcore_axis_name="s")
_SC_PARAMS = pltpu.CompilerParams(use_tc_tiling_on_sc=False)


def _make_sc_agg(compute_deg: bool):
    depth = 3 if compute_deg else 4
    out_type = [jax.ShapeDtypeStruct((NC, N_PAD, D), jnp.float32)]
    scratch = (
        [pltpu.VMEM_SHARED((N_PAD, D), jnp.float32)]
        + [pltpu.VMEM((NCHUNKS, CHUNK), jnp.int32)] * 2
        + [pltpu.VMEM((CHUNK, D), jnp.float32)] * depth
        + [pltpu.SemaphoreType.DMA] * (2 * depth)
    )
    if compute_deg:
        out_type.append(jax.ShapeDtypeStruct((NC, N_PAD, DEG_W), jnp.bfloat16))
        scratch = scratch + [
            pltpu.VMEM_SHARED((N_PAD, DEG_W), jnp.bfloat16),
            pltpu.VMEM((CHUNK, DEG_W), jnp.bfloat16),
            pltpu.SemaphoreType.DMA,
        ]

    @functools.partial(pl.kernel, mesh=_MESH, out_type=out_type,
                       scratch_types=scratch, compiler_params=_SC_PARAMS)
    def sc_agg(*refs):
        if compute_deg:
            (x_hbm, src_hbm, dst_hbm, zrow_hbm, zdeg_hbm, ones_hbm,
             acc_out, deg_out, acc_sh, src_v, dst_v, *rest) = refs
            rows = rest[:depth]
            gsem = rest[depth:2 * depth]
            ssem = rest[2 * depth:3 * depth]
            deg_sh, ones_v, dsem = rest[3 * depth:]
        else:
            (x_hbm, src_hbm, dst_hbm, zrow_hbm,
             acc_out, acc_sh, src_v, dst_v, *rest) = refs
            rows = rest[:depth]
            gsem = rest[depth:2 * depth]
            ssem = rest[2 * depth:]
        cid = lax.axis_index("c")
        sid = lax.axis_index("s")
        w = cid * NS + sid
        sub_rows = pl.ds(sid * ROWS_PER_SUB, ROWS_PER_SUB)
        pltpu.sync_copy(zrow_hbm, acc_sh.at[sub_rows])
        if compute_deg:
            pltpu.sync_copy(zdeg_hbm, deg_sh.at[sub_rows])
            pltpu.sync_copy(ones_hbm, ones_v)
        pltpu.sync_copy(src_hbm.at[w], src_v)
        pltpu.sync_copy(dst_hbm.at[w], dst_v)
        plsc.subcore_barrier()

        def gstart(c, b):
            pltpu.async_copy(x_hbm.at[src_v.at[c]], rows[b], gsem[b])

        def gwait(b):
            pltpu.make_async_copy(x_hbm.at[src_v.at[0]], rows[b], gsem[b]).wait()

        def sstart(c, b):
            pltpu.async_copy(rows[b], acc_sh.at[dst_v.at[c]], ssem[b], add=True)

        def swait(b):
            pltpu.make_async_copy(rows[b], acc_sh.at[dst_v.at[0]], ssem[b]).wait()

        def dstart(c):
            pltpu.async_copy(ones_v, deg_sh.at[dst_v.at[c]], dsem, add=True)

        def dwait():
            pltpu.make_async_copy(ones_v, deg_sh.at[dst_v.at[0]], dsem).wait()

        for b in range(depth):
            gstart(b, b)

        @pl.loop(0, NCHUNKS // depth)
        def _(k):
            c0 = depth * k
            for b in range(depth):
                gwait(b)
                sstart(c0 + b, b)
                if compute_deg:
                    if b == 0:
                        @pl.when(k > 0)
                        def _():
                            dwait()
                    else:
                        dwait()
                    dstart(c0 + b)
            for b in range(depth):
                swait(b)

                @pl.when(c0 + b + depth < NCHUNKS)
                def _():
                    gstart(c0 + b + depth, b)

        if compute_deg:
            dwait()
        plsc.subcore_barrier()
        pltpu.sync_copy(acc_sh.at[sub_rows], acc_out.at[cid].at[sub_rows])
        if compute_deg:
            pltpu.sync_copy(deg_sh.at[sub_rows], deg_out.at[cid].at[sub_rows])

    return sc_agg


_sc_agg_deg = _make_sc_agg(True)
_sc_agg = _make_sc_agg(False)

BLK = 1000


def _combine_body(relu, acc_ref, deg_ref, h_ref, wl_ref, wr_ref, b_ref, out_ref):
    agg = acc_ref[0] + acc_ref[1]
    deg = (deg_ref[0, :, 0:1] + deg_ref[1, :, 0:1]).astype(jnp.float32)
    mean = agg / jnp.maximum(deg, 1.0)
    dn = (((1,), (1,)), ((), ()))
    out = (lax.dot_general(mean, wl_ref[...], dn,
                           preferred_element_type=jnp.float32,
                           precision=lax.Precision.HIGHEST)
           + lax.dot_general(h_ref[...], wr_ref[...], dn,
                             preferred_element_type=jnp.float32,
                             precision=lax.Precision.HIGHEST)
           + b_ref[...])
    out_ref[...] = jnp.maximum(out, 0.0) if relu else out


def _combine(acc, deg, h, wl, wr, b, relu):
    return pl.pallas_call(
        functools.partial(_combine_body, relu),
        grid=(N_NODES // BLK,),
        in_specs=[
            pl.BlockSpec((NC, BLK, D), lambda i: (0, i, 0)),
            pl.BlockSpec((NC, BLK, DEG_W), lambda i: (0, i, 0)),
            pl.BlockSpec((BLK, D), lambda i: (i, 0)),
            pl.BlockSpec((D, D), lambda i: (0, 0)),
            pl.BlockSpec((D, D), lambda i: (0, 0)),
            pl.BlockSpec((1, D), lambda i: (0, 0)),
        ],
        out_specs=pl.BlockSpec((BLK, D), lambda i: (i, 0)),
        out_shape=jax.ShapeDtypeStruct((N_NODES, D), jnp.float32),
    )(acc, deg, h, wl, wr, b.reshape(1, D))


def kernel(x, edge_index, Wl0, Wr0, b0, Wl1, Wr1, b1, Wl2, Wr2, b2):
    src = edge_index[0].astype(jnp.int32)
    dst = edge_index[1].astype(jnp.int32)
    e = src.shape[0]
    pad = E_PAD - e
    pad_idx = jnp.arange(pad, dtype=jnp.int32)
    src3 = jnp.concatenate([src, pad_idx % N_NODES]).reshape(NW, NCHUNKS, CHUNK)
    dst3 = jnp.concatenate(
        [dst, N_NODES + pad_idx % (N_PAD - N_NODES)]).reshape(NW, NCHUNKS, CHUNK)
    zrow = jnp.zeros((ROWS_PER_SUB, D), jnp.float32)
    zdeg = jnp.zeros((ROWS_PER_SUB, DEG_W), jnp.bfloat16)
    ones = jnp.ones((CHUNK, DEG_W), jnp.bfloat16)

    acc, deg = _sc_agg_deg(x, src3, dst3, zrow, zdeg, ones)
    h = _combine(acc, deg, x, Wl0, Wr0, b0, relu=True)
    acc, = _sc_agg(h, src3, dst3, zrow)
    h = _combine(acc, deg, h, Wl1, Wr1, b1, relu=True)
    acc, = _sc_agg(h, src3, dst3, zrow)
    return _combine(acc, deg, h, Wl2, Wr2, b2, relu=False)


# speedup vs baseline: 1.0119x; 1.0119x over previous
import functools

import jax
import jax.numpy as jnp
from jax import lax
from jax.experimental import pallas as pl
from jax.experimental.pallas import tpu as pltpu
from jax.experimental.pallas import tpu_sc as plsc

N_NODES = 10000
D = 128
NC = 2
NS = 16
NW = NC * NS
CHUNK = 56
NCHUNKS = 180
E_PAD = NW * NCHUNKS * CHUNK
N_PAD = 10112
ROWS_PER_SUB = N_PAD // NS
DEG_W = 16

_MESH = plsc.VectorSubcoreMesh(core_axis_name="c", subcore_axis_name="s")
_SC_PARAMS = pltpu.CompilerParams(use_tc_tiling_on_sc=False)


def _make_sc_agg(compute_deg: bool):
    depth = 3 if compute_deg else 4
    out_type = [jax.ShapeDtypeStruct((NC, N_PAD, D), jnp.float32)]
    scratch = (
        [pltpu.VMEM_SHARED((N_PAD, D), jnp.float32)]
        + [pltpu.VMEM((NCHUNKS, CHUNK), jnp.int32)] * 2
        + [pltpu.VMEM((CHUNK, D), jnp.float32)] * depth
        + [pltpu.SemaphoreType.DMA] * (2 * depth)
    )
    if compute_deg:
        out_type.append(jax.ShapeDtypeStruct((NC, N_PAD, DEG_W), jnp.bfloat16))
        scratch = scratch + [
            pltpu.VMEM_SHARED((N_PAD, DEG_W), jnp.bfloat16),
            pltpu.VMEM((CHUNK, DEG_W), jnp.bfloat16),
            pltpu.SemaphoreType.DMA,
        ]

    @functools.partial(pl.kernel, mesh=_MESH, out_type=out_type,
                       scratch_types=scratch, compiler_params=_SC_PARAMS)
    def sc_agg(*refs):
        if compute_deg:
            (x_hbm, src_hbm, dst_hbm, zrow_hbm, zdeg_hbm, ones_hbm,
             acc_out, deg_out, acc_sh, src_v, dst_v, *rest) = refs
            rows = rest[:depth]
            gsem = rest[depth:2 * depth]
            ssem = rest[2 * depth:3 * depth]
            deg_sh, ones_v, dsem = rest[3 * depth:]
        else:
            (x_hbm, src_hbm, dst_hbm, zrow_hbm,
             acc_out, acc_sh, src_v, dst_v, *rest) = refs
            rows = rest[:depth]
            gsem = rest[depth:2 * depth]
            ssem = rest[2 * depth:]
        cid = lax.axis_index("c")
        sid = lax.axis_index("s")
        w = cid * NS + sid
        sub_rows = pl.ds(sid * ROWS_PER_SUB, ROWS_PER_SUB)
        cps = [pltpu.async_copy(zrow_hbm, acc_sh.at[sub_rows], gsem[0]),
               pltpu.async_copy(src_hbm.at[w], src_v, gsem[1]),
               pltpu.async_copy(dst_hbm.at[w], dst_v, gsem[2])]
        if compute_deg:
            cps.append(pltpu.async_copy(zdeg_hbm, deg_sh.at[sub_rows], dsem))
            cps.append(pltpu.async_copy(ones_hbm, ones_v, ssem[0]))
        for cp in cps:
            cp.wait()
        plsc.subcore_barrier()

        def gstart(c, b):
            pltpu.async_copy(x_hbm.at[src_v.at[c]], rows[b], gsem[b])

        def gwait(b):
            pltpu.make_async_copy(x_hbm.at[src_v.at[0]], rows[b], gsem[b]).wait()

        def sstart(c, b):
            pltpu.async_copy(rows[b], acc_sh.at[dst_v.at[c]], ssem[b], add=True)

        def swait(b):
            pltpu.make_async_copy(rows[b], acc_sh.at[dst_v.at[0]], ssem[b]).wait()

        def dstart(c):
            pltpu.async_copy(ones_v, deg_sh.at[dst_v.at[c]], dsem, add=True)

        def dwait():
            pltpu.make_async_copy(ones_v, deg_sh.at[dst_v.at[0]], dsem).wait()

        for b in range(depth):
            gstart(b, b)

        @pl.loop(0, NCHUNKS // depth)
        def _(k):
            c0 = depth * k
            for b in range(depth):
                gwait(b)
                sstart(c0 + b, b)
                if compute_deg:
                    if b == 0:
                        @pl.when(k > 0)
                        def _():
                            dwait()
                    else:
                        dwait()
                    dstart(c0 + b)
            for b in range(depth):
                swait(b)

                @pl.when(c0 + b + depth < NCHUNKS)
                def _():
                    gstart(c0 + b + depth, b)

        if compute_deg:
            dwait()
        plsc.subcore_barrier()
        pltpu.sync_copy(acc_sh.at[sub_rows], acc_out.at[cid].at[sub_rows])
        if compute_deg:
            pltpu.sync_copy(deg_sh.at[sub_rows], deg_out.at[cid].at[sub_rows])

    return sc_agg


_sc_agg_deg = _make_sc_agg(True)
_sc_agg = _make_sc_agg(False)

BLK = 1000


def _combine_body(relu, acc_ref, deg_ref, h_ref, wl_ref, wr_ref, b_ref, out_ref):
    agg = acc_ref[0] + acc_ref[1]
    deg = (deg_ref[0, :, 0:1] + deg_ref[1, :, 0:1]).astype(jnp.float32)
    mean = agg / jnp.maximum(deg, 1.0)
    dn = (((1,), (1,)), ((), ()))
    out = (lax.dot_general(mean, wl_ref[...], dn,
                           preferred_element_type=jnp.float32,
                           precision=lax.Precision.HIGHEST)
           + lax.dot_general(h_ref[...], wr_ref[...], dn,
                             preferred_element_type=jnp.float32,
                             precision=lax.Precision.HIGHEST)
           + b_ref[...])
    out_ref[...] = jnp.maximum(out, 0.0) if relu else out


def _combine(acc, deg, h, wl, wr, b, relu):
    return pl.pallas_call(
        functools.partial(_combine_body, relu),
        grid=(N_NODES // BLK,),
        in_specs=[
            pl.BlockSpec((NC, BLK, D), lambda i: (0, i, 0)),
            pl.BlockSpec((NC, BLK, DEG_W), lambda i: (0, i, 0)),
            pl.BlockSpec((BLK, D), lambda i: (i, 0)),
            pl.BlockSpec((D, D), lambda i: (0, 0)),
            pl.BlockSpec((D, D), lambda i: (0, 0)),
            pl.BlockSpec((1, D), lambda i: (0, 0)),
        ],
        out_specs=pl.BlockSpec((BLK, D), lambda i: (i, 0)),
        out_shape=jax.ShapeDtypeStruct((N_NODES, D), jnp.float32),
    )(acc, deg, h, wl, wr, b.reshape(1, D))


def kernel(x, edge_index, Wl0, Wr0, b0, Wl1, Wr1, b1, Wl2, Wr2, b2):
    src = edge_index[0].astype(jnp.int32)
    dst = edge_index[1].astype(jnp.int32)
    e = src.shape[0]
    pad = E_PAD - e
    pad_idx = jnp.arange(pad, dtype=jnp.int32)
    src3 = jnp.concatenate([src, pad_idx % N_NODES]).reshape(NW, NCHUNKS, CHUNK)
    dst3 = jnp.concatenate(
        [dst, N_NODES + pad_idx % (N_PAD - N_NODES)]).reshape(NW, NCHUNKS, CHUNK)
    zrow = jnp.zeros((ROWS_PER_SUB, D), jnp.float32)
    zdeg = jnp.zeros((ROWS_PER_SUB, DEG_W), jnp.bfloat16)
    ones = jnp.ones((CHUNK, DEG_W), jnp.bfloat16)

    acc, deg = _sc_agg_deg(x, src3, dst3, zrow, zdeg, ones)
    h = _combine(acc, deg, x, Wl0, Wr0, b0, relu=True)
    acc, = _sc_agg(h, src3, dst3, zrow)
    h = _combine(acc, deg, h, Wl1, Wr1, b1, relu=True)
    acc, = _sc_agg(h, src3, dst3, zrow)
    return _combine(acc, deg, h, Wl2, Wr2, b2, relu=False)
